# baseline (device time: 77565 ns/iter reference)
import jax
import jax.numpy as jnp
from jax import lax
from jax.experimental import pallas as pl
from jax.experimental.pallas import tpu as pltpu

N_DEV = 8

ORDERINGS = (
    (0, 640, (1, 3, 4)),
    (640, 704, (3, 4, 1)),
    (1344, 704, (4, 1, 3)),
)


def _kept(masks, k):
    m1, m2, m3 = masks
    if k == 0:
        return [m2, m2 ^ m3, m3, 0]
    if k == 1:
        return [m3, 0]
    return [0]


_SLOT_BASE = (0, 4, 6)


def kernel(x, w_mat, scale_x, scale_w):
    m_tot, k_per = x.shape
    _, n = w_mat.shape
    m_per = m_tot // N_DEV

    def body(x_ref, w_ref, sx_ref, sw_ref, out_ref,
             p_a, p_b, p_c, r_a, r_b, r_c,
             ssem_a, rsem_a, ssem_b, rsem_b, ssem_c, rsem_c):
        my = lax.axis_index("i")
        pbufs = (p_a, p_b, p_c)
        rbufs = (r_a, r_b, r_c)
        ssems = (ssem_a, ssem_b, ssem_c)
        rsems = (rsem_a, rsem_b, rsem_c)

        barrier_sem = pltpu.get_barrier_semaphore()
        for mask in (1, 3, 4):
            pl.semaphore_signal(
                barrier_sem, inc=1,
                device_id=(my ^ mask,), device_id_type=pl.DeviceIdType.MESH,
            )
        pl.semaphore_wait(barrier_sem, 3)

        w_bf = w_ref[:, :].astype(jnp.bfloat16)

        def compute_chunk(o, c):
            c0, cw, _ = ORDERINGS[o]
            xc = x_ref[pl.ds(c * m_per, m_per), :].astype(jnp.bfloat16)
            part = jax.lax.dot(
                xc, w_bf[:, c0:c0 + cw], preferred_element_type=jnp.float32)
            pbufs[o][c] = part.astype(jnp.bfloat16)

        pending = []

        def start_msg(o, k, j):
            masks = ORDERINGS[o][2]
            t = _kept(masks, k)[j]
            slot = _SLOT_BASE[k] + j
            rdma = pltpu.make_async_remote_copy(
                src_ref=pbufs[o].at[my ^ (t ^ masks[k])],
                dst_ref=rbufs[o].at[slot],
                send_sem=ssems[o].at[slot],
                recv_sem=rsems[o].at[slot],
                device_id=(my ^ masks[k],),
                device_id_type=pl.DeviceIdType.MESH,
            )
            rdma.start()
            pending.append(rdma)
            return rdma

        rd = {}

        for j in range(4):
            for o in range(3):
                masks = ORDERINGS[o][2]
                t = _kept(masks, 0)[j]
                compute_chunk(o, my ^ (t ^ masks[0]))
                rd[(o, 0, j)] = start_msg(o, 0, j)

        for t_i in range(4):
            for o in range(3):
                masks = ORDERINGS[o][2]
                compute_chunk(o, my ^ _kept(masks, 0)[t_i])

        def add_msg(o, k, j):
            masks = ORDERINGS[o][2]
            t = _kept(masks, k)[j]
            slot = _SLOT_BASE[k] + j
            rd[(o, k, j)].wait_recv()
            c = my ^ t
            pbufs[o][c] = pbufs[o][c] + rbufs[o][slot]

        for o in range(3):
            add_msg(o, 0, 0)
        for o in range(3):
            add_msg(o, 0, 1)
            rd[(o, 1, 0)] = start_msg(o, 1, 0)
            rd[(o, 1, 1)] = start_msg(o, 1, 1)
        for o in range(3):
            add_msg(o, 0, 2)
        for o in range(3):
            add_msg(o, 1, 0)
            rd[(o, 2, 0)] = start_msg(o, 2, 0)
        for o in range(3):
            add_msg(o, 0, 3)
        for o in range(3):
            add_msg(o, 1, 1)

        scale = sx_ref[0] * sw_ref[0]
        for o in range(3):
            c0, cw, _ = ORDERINGS[o]
            rd[(o, 2, 0)].wait_recv()
            acc = (pbufs[o][my].astype(jnp.float32)
                   + rbufs[o][6].astype(jnp.float32))
            out_ref[:, c0:c0 + cw] = jnp.maximum(acc * scale, 0.0)

        for rdma in pending:
            rdma.wait_send()

    scratch_shapes = (
        [pltpu.VMEM((N_DEV, m_per, cw), jnp.bfloat16) for _, cw, _m in ORDERINGS]
        + [pltpu.VMEM((7, m_per, cw), jnp.bfloat16) for _, cw, _m in ORDERINGS]
    )
    for _ in range(3):
        scratch_shapes.append(pltpu.SemaphoreType.DMA((7,)))
        scratch_shapes.append(pltpu.SemaphoreType.DMA((7,)))

    return pl.pallas_call(
        body,
        out_shape=jax.ShapeDtypeStruct((m_per, n), jnp.float32),
        in_specs=[
            pl.BlockSpec(memory_space=pltpu.VMEM),
            pl.BlockSpec(memory_space=pltpu.VMEM),
            pl.BlockSpec(memory_space=pltpu.SMEM),
            pl.BlockSpec(memory_space=pltpu.SMEM),
        ],
        out_specs=pl.BlockSpec(memory_space=pltpu.VMEM),
        scratch_shapes=scratch_shapes,
        compiler_params=pltpu.CompilerParams(
            collective_id=0, vmem_limit_bytes=100 * 1024 * 1024),
    )(x, w_mat, scale_x, scale_w)
